# Initial kernel scaffold; baseline (speedup 1.0000x reference)
import jax
import jax.numpy as jnp
from jax.experimental import pallas as pl


def kernel(x, edge_index, set_indices, batch, num_graphs, Wl1, bl1, Wr1, Wl2, bl2, Wr2, Wm, bm, W1, b1, W2, b2):
    raise NotImplementedError("write your pallas kernel here")


# speedup vs baseline: 6.4030x; 6.4030x over previous
import functools
import jax
import jax.numpy as jnp
from jax import lax
from jax.experimental import pallas as pl
from jax.experimental.pallas import tpu as pltpu
from jax.experimental.pallas import tpu_sc as plsc

N = 10000
NPAD = 10240
E = 320000
K = 128
NCHUNK = E // K
NC, NS = 2, 16
NW = NC * NS
ROWS_PER_SUB = NPAD // NS


def _sc_body(with_deg, x_hbm, src_hbm, dst_hbm, zrow_hbm, zone_hbm,
             *refs):
    if with_deg:
        (acc_out, deg_out, idx_s, idx_d, rows_v, ones_v, sem,
         acc_sh, deg_sh) = refs
    else:
        (acc_out, idx_s, idx_d, rows_v, sem, acc_sh) = refs
    c = lax.axis_index("c")
    s = lax.axis_index("s")
    w = s * NC + c

    pltpu.sync_copy(zrow_hbm, acc_sh.at[pl.ds(s * ROWS_PER_SUB, ROWS_PER_SUB)])
    if with_deg:
        pltpu.sync_copy(zone_hbm, deg_sh.at[pl.ds(s * ROWS_PER_SUB, ROWS_PER_SUB)])
        for j in range(K // 16):
            ones_v[pl.ds(j * 16, 16)] = jnp.ones((16,), jnp.float32)
    plsc.subcore_barrier()

    def chunk(t, carry):
        cid = w + NW * t

        @pl.when(cid < NCHUNK)
        def _():
            pltpu.sync_copy(src_hbm.at[cid], idx_s)
            pltpu.sync_copy(dst_hbm.at[cid], idx_d)
            pltpu.async_copy(x_hbm.at[idx_s], rows_v, sem).wait()
            pltpu.sync_copy(rows_v, acc_sh.at[idx_d], add=True)
            if with_deg:
                pltpu.sync_copy(ones_v, deg_sh.at[idx_d], add=True)
        return carry

    nt = NCHUNK // NW + 1
    lax.fori_loop(0, nt, chunk, 0)
    plsc.subcore_barrier()

    sl = pl.ds(s * ROWS_PER_SUB, ROWS_PER_SUB)
    pltpu.sync_copy(acc_sh.at[sl], acc_out.at[c, sl])
    if with_deg:
        pltpu.sync_copy(deg_sh.at[sl], deg_out.at[c, sl])


def _make_sc_call(with_deg):
    out_type = [jax.ShapeDtypeStruct((NC, NPAD, 128), jnp.float32)]
    scratch = [
        pltpu.VMEM((K,), jnp.int32),
        pltpu.VMEM((K,), jnp.int32),
        pltpu.VMEM((K, 128), jnp.float32),
    ]
    if with_deg:
        out_type.append(jax.ShapeDtypeStruct((NC, NPAD), jnp.float32))
        scratch.append(pltpu.VMEM((K,), jnp.float32))
    scratch.append(pltpu.SemaphoreType.DMA)
    scratch.append(pltpu.VMEM_SHARED((NPAD, 128), jnp.float32))
    if with_deg:
        scratch.append(pltpu.VMEM_SHARED((NPAD,), jnp.float32))
    mesh = plsc.VectorSubcoreMesh(core_axis_name="c", subcore_axis_name="s",
                                  num_cores=NC, num_subcores=NS)
    return pl.kernel(
        functools.partial(_sc_body, with_deg),
        out_type=tuple(out_type),
        mesh=mesh,
        scratch_types=tuple(scratch),
        name="sage_segsum_sc" + ("_deg" if with_deg else ""),
    )


def _dense_body(a0, a1, d0, d1, xb, WlT, bl, WrT, out):
    deg = jnp.maximum(d0[...] + d1[...], 1.0)
    agg = (a0[...] + a1[...]) / deg
    h = (jnp.dot(agg, WlT[...], preferred_element_type=jnp.float32)
         + bl[...]
         + jnp.dot(xb[...], WrT[...], preferred_element_type=jnp.float32))
    out[...] = jnp.maximum(h, 0.0)


BR = 1280


def _dense_call(a0, a1, d0, d1, xb, WlT, bl, WrT):
    nblk = NPAD // BR
    row = lambda i: (i, 0)
    fixed = lambda i: (0, 0)
    return pl.pallas_call(
        _dense_body,
        grid=(nblk,),
        in_specs=[
            pl.BlockSpec((BR, 128), row),
            pl.BlockSpec((BR, 128), row),
            pl.BlockSpec((BR, 1), row),
            pl.BlockSpec((BR, 1), row),
            pl.BlockSpec((BR, 128), row),
            pl.BlockSpec((128, 128), fixed),
            pl.BlockSpec((1, 128), fixed),
            pl.BlockSpec((128, 128), fixed),
        ],
        out_specs=pl.BlockSpec((BR, 128), row),
        out_shape=jax.ShapeDtypeStruct((NPAD, 128), jnp.float32),
    )(a0, a1, d0, d1, xb, WlT, bl, WrT)


def _tail_body(h2, batch2d, set01, WmdT, WmmT, WmxT, bm, W1T, b1,
               W2Tp, b2p, out):
    b = batch2d[...]
    g3 = lax.broadcasted_iota(jnp.int32, (128, 80, 128), 0)
    cmp = (b[None, :, :] < g3).astype(jnp.int32)
    base = jnp.sum(jnp.sum(cmp, axis=2), axis=1, keepdims=True)
    idx0 = jnp.clip(base + set01[:, 0:1], 0, N - 1)
    idx1 = jnp.clip(base + set01[:, 1:2], 0, N - 1)
    col = lax.broadcasted_iota(jnp.int32, (128, NPAD), 1)
    h = h2[...]
    xs0 = jnp.dot((col == idx0).astype(jnp.float32), h,
                  preferred_element_type=jnp.float32)
    xs1 = jnp.dot((col == idx1).astype(jnp.float32), h,
                  preferred_element_type=jnp.float32)
    d = jnp.abs(xs0 - xs1)
    m = (xs0 + xs1) * 0.5
    x = jnp.maximum(xs0, xs1)
    pooled = (jnp.dot(d, WmdT[...], preferred_element_type=jnp.float32)
              + jnp.dot(m, WmmT[...], preferred_element_type=jnp.float32)
              + jnp.dot(x, WmxT[...], preferred_element_type=jnp.float32)
              + bm[...])
    f = jnp.maximum(
        jnp.dot(pooled, W1T[...], preferred_element_type=jnp.float32) + b1[...],
        0.0)
    logits = jnp.dot(f, W2Tp[...], preferred_element_type=jnp.float32) + b2p[...]
    mx = jnp.max(logits, axis=1, keepdims=True)
    lse = jnp.log(jnp.sum(jnp.exp(logits - mx), axis=1, keepdims=True))
    out[...] = logits - mx - lse


def _tail_call(h2, batch2d, set01, WmdT, WmmT, WmxT, bm, W1T, b1, W2Tp, b2p):
    return pl.pallas_call(
        _tail_body,
        out_shape=jax.ShapeDtypeStruct((128, 128), jnp.float32),
    )(h2, batch2d, set01, WmdT, WmmT, WmxT, bm, W1T, b1, W2Tp, b2p)


def kernel(x, edge_index, set_indices, batch, num_graphs,
           Wl1, bl1, Wr1, Wl2, bl2, Wr2, Wm, bm, W1, b1, W2, b2):
    del num_graphs
    f32 = jnp.float32

    xp = jnp.pad(x, ((0, NPAD - N), (0, 0)))
    src2d = edge_index[0].reshape(NCHUNK, K)
    dst2d = edge_index[1].reshape(NCHUNK, K)
    zrow = jnp.zeros((ROWS_PER_SUB, 128), f32)
    zone = jnp.zeros((ROWS_PER_SUB,), f32)
    batch2d = jnp.pad(batch, (0, NPAD - N), constant_values=127).reshape(80, 128)
    set01 = jnp.pad(set_indices, ((0, 128 - set_indices.shape[0]), (0, 6)))
    Wl1T, Wr1T = Wl1.T, Wr1.T
    Wl2T, Wr2T = Wl2.T, Wr2.T
    bl1r, bl2r = bl1.reshape(1, 128), bl2.reshape(1, 128)
    WmdT = Wm[:, 0:128].T
    WmmT = Wm[:, 128:256].T
    WmxT = Wm[:, 256:384].T
    bmr = bm.reshape(1, 128)
    W1T = W1.T
    b1r = b1.reshape(1, 128)
    W2Tp = jnp.pad(W2.T, ((0, 0), (0, 128 - W2.shape[0])))
    b2p = jnp.pad(b2, (0, 128 - W2.shape[0]),
                  constant_values=-1e30).reshape(1, 128)

    acc1, deg = _make_sc_call(True)(xp, src2d, dst2d, zrow, zone)
    d0 = deg[0].reshape(NPAD, 1)
    d1 = deg[1].reshape(NPAD, 1)
    h1 = _dense_call(acc1[0], acc1[1], d0, d1, xp, Wl1T, bl1r, Wr1T)

    acc2 = _make_sc_call(False)(h1, src2d, dst2d, zrow, zone)[0]
    h2 = _dense_call(acc2[0], acc2[1], d0, d1, h1, Wl2T, bl2r, Wr2T)

    outp = _tail_call(h2, batch2d, set01, WmdT, WmmT, WmxT, bmr,
                      W1T, b1r, W2Tp, b2p)
    return outp[:set_indices.shape[0], :W2.shape[0]]
